# baseline (speedup 1.0000x reference)
import functools

import jax
import jax.numpy as jnp
from jax import lax
from jax.experimental import pallas as pl
from jax.experimental.pallas import tpu as pltpu
from jax.experimental.pallas import tpu_sc as plsc

_B = 16384
_D = 768
_V = 12

_info = plsc.get_sparse_core_info()
_NC = _info.num_cores
_NS = _info.num_subcores
_NW = _NC * _NS
_BPW = _B // _NW
_CHUNK = 64
_NBUF = 2
_NCHUNK = _BPW // _CHUNK
_L = 16

_mesh = plsc.VectorSubcoreMesh(core_axis_name="c", subcore_axis_name="s")


@functools.partial(
    pl.kernel,
    mesh=_mesh,
    out_type=jax.ShapeDtypeStruct((_B, _D), jnp.float32),
    scratch_types=[
        pltpu.VMEM((_BPW,), jnp.int32),
        pltpu.VMEM((_V, _D), jnp.float32),
        pltpu.VMEM((_NBUF, _CHUNK, _D), jnp.float32),
        pltpu.SemaphoreType.DMA,
    ],
    compiler_params=pltpu.CompilerParams(
        use_tc_tiling_on_sc=False, needs_layout_passes=False
    ),
)
def _emb_lookup(labels_hbm, table_hbm, out_hbm, idx_v, table_v, obuf, ssem):
    wid = lax.axis_index("s") * _NC + lax.axis_index("c")
    base = wid * _BPW
    pltpu.sync_copy(table_hbm, table_v)
    pltpu.sync_copy(labels_hbm.at[pl.ds(base, _BPW)], idx_v)

    iota = lax.iota(jnp.int32, _L)
    col0 = jnp.zeros((_L,), jnp.int32)

    def scatter(c):
        return pltpu.make_async_copy(
            obuf.at[c % _NBUF],
            out_hbm.at[pl.ds(base + c * _CHUNK, _CHUNK)],
            ssem,
        )

    for c in range(_NCHUNK):
        buf = c % _NBUF
        if c >= _NBUF:
            scatter(c - _NBUF).wait()
        for g in range(_CHUNK // _L):
            lab16 = idx_v[pl.ds(c * _CHUNK + g * _L, _L)]
            row16 = iota + (g * _L)

            def body(_, col, lab16=lab16, row16=row16, buf=buf):
                vals = plsc.load_gather(table_v, [lab16, col])
                plsc.store_scatter(obuf.at[buf], [row16, col], vals)
                return col + 1

            lax.fori_loop(0, _D, body, col0, unroll=16)
        scatter(c).start()
    scatter(_NCHUNK - 2).wait()
    scatter(_NCHUNK - 1).wait()


def kernel(labels, table):
    return _emb_lookup(labels.astype(jnp.int32), table)


# speedup vs baseline: 3.0658x; 3.0658x over previous
import functools

import jax
import jax.numpy as jnp
from jax import lax
from jax.experimental import pallas as pl
from jax.experimental.pallas import tpu as pltpu
from jax.experimental.pallas import tpu_sc as plsc

_B = 16384
_D = 768
_V = 12

_info = plsc.get_sparse_core_info()
_NC = _info.num_cores
_NS = _info.num_subcores
_NW = _NC * _NS
_BPW = _B // _NW
_CHUNK = 64
_NBUF = 2
_NCHUNK = _BPW // _CHUNK
_L = 16

_mesh = plsc.VectorSubcoreMesh(core_axis_name="c", subcore_axis_name="s")


@functools.partial(
    pl.kernel,
    mesh=_mesh,
    out_type=jax.ShapeDtypeStruct((_B * _D,), jnp.float32),
    scratch_types=[
        pltpu.VMEM((_BPW,), jnp.int32),
        pltpu.VMEM((_V * _D,), jnp.float32),
        pltpu.VMEM((_NBUF, _CHUNK * _D), jnp.float32),
        pltpu.SemaphoreType.DMA,
    ],
    compiler_params=pltpu.CompilerParams(
        use_tc_tiling_on_sc=False, needs_layout_passes=False
    ),
)
def _emb_lookup(labels_hbm, table_hbm, out_hbm, idx_v, table_v, obuf, ssem):
    wid = lax.axis_index("s") * _NC + lax.axis_index("c")
    base = wid * _BPW
    pltpu.sync_copy(table_hbm, table_v)
    pltpu.sync_copy(labels_hbm.at[pl.ds(base, _BPW)], idx_v)

    iota = lax.iota(jnp.int32, _L)

    def scatter(c):
        return pltpu.make_async_copy(
            obuf.at[c % _NBUF],
            out_hbm.at[pl.ds((base + c * _CHUNK) * _D, _CHUNK * _D)],
            ssem,
        )

    for c in range(_NCHUNK):
        buf = c % _NBUF
        if c >= _NBUF:
            scatter(c - _NBUF).wait()

        def row_body(_, carry, buf=buf):
            rsplat, didx = carry
            labsp = plsc.load_gather(idx_v, [rsplat])
            gidx = labsp * _D + iota
            for _k in range(_D // _L):
                vals = plsc.load_gather(table_v, [gidx])
                plsc.store_scatter(obuf.at[buf], [didx], vals)
                gidx = gidx + _L
                didx = didx + _L
            return rsplat + 1, didx

        r0 = jnp.full((_L,), c * _CHUNK, jnp.int32)
        lax.fori_loop(0, _CHUNK, row_body, (r0, iota))
        scatter(c).start()
    scatter(_NCHUNK - 2).wait()
    scatter(_NCHUNK - 1).wait()


def kernel(labels, table):
    out = _emb_lookup(labels.astype(jnp.int32), table.reshape(-1))
    return out.reshape(_B, _D)


# speedup vs baseline: 4.7285x; 1.5423x over previous
import functools

import jax
import jax.numpy as jnp
from jax import lax
from jax.experimental import pallas as pl
from jax.experimental.pallas import tpu as pltpu
from jax.experimental.pallas import tpu_sc as plsc

_B = 16384
_D = 768
_V = 12

_info = plsc.get_sparse_core_info()
_NC = _info.num_cores
_NS = _info.num_subcores
_NW = _NC * _NS
_BPW = _B // _NW
_CHUNK = 64
_NBUF = 2
_NCHUNK = _BPW // _CHUNK
_L = 16

_mesh = plsc.VectorSubcoreMesh(core_axis_name="c", subcore_axis_name="s")


@functools.partial(
    pl.kernel,
    mesh=_mesh,
    out_type=jax.ShapeDtypeStruct((_B * _D,), jnp.float32),
    scratch_types=[
        pltpu.VMEM((_BPW + _L,), jnp.int32),
        pltpu.VMEM((_V * _D,), jnp.float32),
        pltpu.VMEM((_NBUF, _CHUNK * _D), jnp.float32),
        pltpu.SemaphoreType.DMA,
    ],
    compiler_params=pltpu.CompilerParams(
        use_tc_tiling_on_sc=False, needs_layout_passes=False
    ),
)
def _emb_lookup(labels_hbm, table_hbm, out_hbm, idx_v, table_v, obuf, ssem):
    wid = lax.axis_index("s") * _NC + lax.axis_index("c")
    base = wid * _BPW
    pltpu.sync_copy(table_hbm, table_v)
    pltpu.sync_copy(labels_hbm.at[pl.ds(base, _BPW)], idx_v.at[pl.ds(0, _BPW)])

    iota = lax.iota(jnp.int32, _L)

    def scatter(c):
        return pltpu.make_async_copy(
            obuf.at[c % _NBUF],
            out_hbm.at[pl.ds((base + c * _CHUNK) * _D, _CHUNK * _D)],
            ssem,
        )

    for c in range(_NCHUNK):
        buf = c % _NBUF
        if c >= _NBUF:
            scatter(c - _NBUF).wait()

        def row_body(r, didx, buf=buf):
            lab = idx_v[pl.ds(r, _L)][0]
            gb = lab * _D
            for k0 in range(0, _D // _L, 12):
                vals = [
                    table_v[pl.ds(gb + (k0 + k) * _L, _L)]
                    for k in range(12)
                ]
                for k in range(12):
                    plsc.store_scatter(obuf.at[buf], [didx], vals[k])
                    didx = didx + _L
            return didx

        lax.fori_loop(c * _CHUNK, (c + 1) * _CHUNK, row_body, iota)
        scatter(c).start()
    scatter(_NCHUNK - 2).wait()
    scatter(_NCHUNK - 1).wait()


def kernel(labels, table):
    out = _emb_lookup(labels.astype(jnp.int32), table.reshape(-1))
    return out.reshape(_B, _D)


# speedup vs baseline: 4.7518x; 1.0049x over previous
import functools

import jax
import jax.numpy as jnp
from jax import lax
from jax.experimental import pallas as pl
from jax.experimental.pallas import tpu as pltpu
from jax.experimental.pallas import tpu_sc as plsc

_B = 16384
_D = 768
_V = 12

_info = plsc.get_sparse_core_info()
_NC = _info.num_cores
_NS = _info.num_subcores
_NW = _NC * _NS
_BPW = _B // _NW
_CHUNK = 64
_NBUF = 2
_NCHUNK = _BPW // _CHUNK
_L = 16

_mesh = plsc.VectorSubcoreMesh(core_axis_name="c", subcore_axis_name="s")


@functools.partial(
    pl.kernel,
    mesh=_mesh,
    out_type=jax.ShapeDtypeStruct((_B * _D,), jnp.float32),
    scratch_types=[
        pltpu.VMEM((_BPW + _L,), jnp.int32),
        pltpu.VMEM((_V * _D,), jnp.float32),
        pltpu.VMEM((_NBUF, _CHUNK * _D), jnp.float32),
        pltpu.SemaphoreType.DMA,
    ],
    compiler_params=pltpu.CompilerParams(
        use_tc_tiling_on_sc=False, needs_layout_passes=False
    ),
)
def _emb_lookup(labels_hbm, table_hbm, out_hbm, idx_v, table_v, obuf, ssem):
    wid = lax.axis_index("s") * _NC + lax.axis_index("c")
    base = wid * _BPW
    pltpu.sync_copy(table_hbm, table_v)
    pltpu.sync_copy(labels_hbm.at[pl.ds(base, _BPW)], idx_v.at[pl.ds(0, _BPW)])

    iota = lax.iota(jnp.int32, _L)

    def scatter(c):
        return pltpu.make_async_copy(
            obuf.at[c % _NBUF],
            out_hbm.at[pl.ds((base + c * _CHUNK) * _D, _CHUNK * _D)],
            ssem,
        )

    for c in range(_NCHUNK):
        buf = c % _NBUF
        if c >= _NBUF:
            scatter(c - _NBUF).wait()

        def row_body(r, _, buf=buf, c=c):
            lab = idx_v[pl.ds(r, _L)][0]
            gb = lab * _D
            db = (r - c * _CHUNK) * _D
            for k0 in range(0, _D // _L, 12):
                vals = [
                    table_v[pl.ds(gb + (k0 + k) * _L, _L)]
                    for k in range(12)
                ]
                for k in range(12):
                    obuf.at[buf][pl.ds(db + (k0 + k) * _L, _L)] = vals[k]
            return 0

        lax.fori_loop(c * _CHUNK, (c + 1) * _CHUNK, row_body, 0)
        scatter(c).start()
    scatter(_NCHUNK - 2).wait()
    scatter(_NCHUNK - 1).wait()


def kernel(labels, table):
    out = _emb_lookup(labels.astype(jnp.int32), table.reshape(-1))
    return out.reshape(_B, _D)


# speedup vs baseline: 5.8134x; 1.2234x over previous
import functools

import jax
import jax.numpy as jnp
from jax import lax
from jax.experimental import pallas as pl
from jax.experimental.pallas import tpu as pltpu
from jax.experimental.pallas import tpu_sc as plsc

_B = 16384
_D = 768
_V = 12

_info = plsc.get_sparse_core_info()
_NC = _info.num_cores
_NS = _info.num_subcores
_NW = _NC * _NS
_BPW = _B // _NW
_L = 16
_Q = 8

_mesh = plsc.VectorSubcoreMesh(core_axis_name="c", subcore_axis_name="s")


@functools.partial(
    pl.kernel,
    mesh=_mesh,
    out_type=jax.ShapeDtypeStruct((_B * _D,), jnp.float32),
    scratch_types=[
        pltpu.VMEM((_BPW + _L,), jnp.int32),
        pltpu.VMEM((_V * _D,), jnp.float32),
        pltpu.SemaphoreType.DMA,
    ],
    compiler_params=pltpu.CompilerParams(
        use_tc_tiling_on_sc=False, needs_layout_passes=False
    ),
)
def _emb_lookup(labels_hbm, table_hbm, out_hbm, idx_v, table_v, ssem):
    wid = lax.axis_index("s") * _NC + lax.axis_index("c")
    base = wid * _BPW
    pltpu.sync_copy(table_hbm, table_v)
    pltpu.sync_copy(labels_hbm.at[pl.ds(base, _BPW)], idx_v.at[pl.ds(0, _BPW)])

    def row_copy(r):
        lab = idx_v[pl.ds(r, _L)][0]
        return pltpu.make_async_copy(
            table_v.at[pl.ds(lab * _D, _D)],
            out_hbm.at[pl.ds((base + r) * _D, _D)],
            ssem,
        )

    for r in range(_Q):
        row_copy(r).start()

    def body(r, _):
        row_copy(r + _Q).start()
        row_copy(r).wait()
        return 0

    lax.fori_loop(0, _BPW - _Q, body, 0)
    for r in range(_BPW - _Q, _BPW):
        row_copy(r).wait()


def kernel(labels, table):
    out = _emb_lookup(labels.astype(jnp.int32), table.reshape(-1))
    return out.reshape(_B, _D)
